# Initial kernel scaffold; baseline (speedup 1.0000x reference)
import functools

import jax
import jax.numpy as jnp
from jax import lax
from jax.experimental import pallas as pl
from jax.experimental.pallas import tpu as pltpu
from jax.experimental.pallas import tpu_sc as plsc

NV, NE, NZ = 10000, 5000, 160000
VP, EP = 10240, 5120
NZP = 161792
CIN, HID = 256, 512
NC, NS = 2, 16
CPT = 79
DPT = 316

f32 = jnp.float32
SDS = jax.ShapeDtypeStruct

_mesh = plsc.VectorSubcoreMesh(
    core_axis_name="c", subcore_axis_name="s", num_cores=NC, num_subcores=NS
)


def _zero_ref(ref, nrows):
    zv = jnp.zeros((16,), f32)

    def body(i, _):
        ref[i] = zv
        return _

    lax.fori_loop(0, nrows, body, None)


@functools.partial(
    pl.kernel,
    out_type=(SDS((32, 640, 16), f32), SDS((32, 320, 16), f32)),
    mesh=_mesh,
    scratch_types=[
        pltpu.VMEM((DPT, 16), jnp.int32),
        pltpu.VMEM((DPT, 16), jnp.int32),
        pltpu.VMEM((640, 16), f32),
        pltpu.VMEM((320, 16), f32),
    ],
)
def _deg1(v3, e3, dvp, cep, vb, eb, dvl, cel):
    c = lax.axis_index("c")
    s = lax.axis_index("s")
    w = s * NC + c
    pltpu.sync_copy(v3.at[w], vb)
    pltpu.sync_copy(e3.at[w], eb)
    _zero_ref(dvl, 640)
    _zero_ref(cel, 320)
    ones = jnp.ones((16,), f32)

    def step(i, _):
        iv = vb[i]
        ie = eb[i]
        plsc.addupdate_scatter(
            dvl, [lax.shift_right_logical(iv, 4), lax.bitwise_and(iv, 15)], ones
        )
        plsc.addupdate_scatter(
            cel, [lax.shift_right_logical(ie, 4), lax.bitwise_and(ie, 15)], ones
        )
        return _

    lax.fori_loop(0, DPT, step, None)
    pltpu.sync_copy(dvl, dvp.at[w])
    pltpu.sync_copy(cel, cep.at[w])


@functools.partial(
    pl.kernel,
    out_type=SDS((32, 320, 16), f32),
    mesh=_mesh,
    scratch_types=[
        pltpu.VMEM((DPT, 16), jnp.int32),
        pltpu.VMEM((DPT, 16), jnp.int32),
        pltpu.VMEM((640, 16), f32),
        pltpu.VMEM((320, 16), f32),
    ],
)
def _deg2(v3, e3, dvf, dep, vb, eb, dvl, del_):
    c = lax.axis_index("c")
    s = lax.axis_index("s")
    w = s * NC + c
    pltpu.sync_copy(v3.at[w], vb)
    pltpu.sync_copy(e3.at[w], eb)
    pltpu.sync_copy(dvf, dvl)
    _zero_ref(del_, 320)

    def step(i, _):
        iv = vb[i]
        ie = eb[i]
        val = plsc.load_gather(
            dvl, [lax.shift_right_logical(iv, 4), lax.bitwise_and(iv, 15)]
        )
        plsc.addupdate_scatter(
            del_, [lax.shift_right_logical(ie, 4), lax.bitwise_and(ie, 15)], val
        )
        return _

    lax.fori_loop(0, DPT, step, None)
    pltpu.sync_copy(del_, dep.at[w])


def _red1_body(dvp, cep, sv, dvf, cnt):
    dv = jnp.sum(dvp[...], axis=0)
    cn = jnp.sum(cep[...], axis=0)
    sv[...] = jnp.where(dv > 0, lax.rsqrt(dv), 0.0)
    dvf[...] = dv
    cnt[...] = cn


_red1 = pl.pallas_call(
    _red1_body,
    out_shape=(SDS((VP,), f32), SDS((VP,), f32), SDS((EP,), f32)),
)


def _red2_body(dep, cnt, q):
    cn = cnt[...]
    cs = jnp.where(cn == 0, 1.0, cn)
    de = jnp.sum(dep[...], axis=0) / cs
    q[...] = jnp.where(de > 0, lax.rsqrt(de), 0.0) / cs


_red2 = pl.pallas_call(_red2_body, out_shape=SDS((EP,), f32))


BM = 1280


def _mm1_body(x, w, b, lo, hi):
    xt = jnp.dot(x[...], w[...], preferred_element_type=f32) + b[...]
    lo[...] = xt[:, :256]
    hi[...] = xt[:, 256:]


_mm1 = pl.pallas_call(
    _mm1_body,
    grid=(VP // BM,),
    in_specs=[
        pl.BlockSpec((BM, CIN), lambda i: (i, 0)),
        pl.BlockSpec((CIN, HID), lambda i: (0, 0)),
        pl.BlockSpec((1, HID), lambda i: (0, 0)),
    ],
    out_specs=(
        pl.BlockSpec((BM, 256), lambda i: (i, 0)),
        pl.BlockSpec((BM, 256), lambda i: (i, 0)),
    ),
    out_shape=(SDS((VP, 256), f32), SDS((VP, 256), f32)),
)


def _mm2_body(o0, o1, o2, o3, sv, w, b, lo, hi):
    h = jnp.concatenate([o0[...], o1[...], o2[...], o3[...]], axis=1)
    h = jnp.maximum(h * sv[...], 0.0)
    xt = jnp.dot(h, w[...], preferred_element_type=f32) + b[...]
    lo[...] = xt[:, :256]
    hi[...] = xt[:, 256:]


_mm2 = pl.pallas_call(
    _mm2_body,
    grid=(VP // BM,),
    in_specs=[pl.BlockSpec((BM, 128), lambda i: (i, 0))] * 4
    + [
        pl.BlockSpec((BM, 1), lambda i: (i, 0)),
        pl.BlockSpec((HID, HID), lambda i: (0, 0)),
        pl.BlockSpec((1, HID), lambda i: (0, 0)),
    ],
    out_specs=(
        pl.BlockSpec((BM, 256), lambda i: (i, 0)),
        pl.BlockSpec((BM, 256), lambda i: (i, 0)),
    ),
    out_shape=(SDS((VP, 256), f32), SDS((VP, 256), f32)),
)


def _fin_body(o0, o1, o2, o3, sv, out):
    h = jnp.concatenate([o0[...], o1[...], o2[...], o3[...]], axis=1)
    out[...] = jnp.maximum(h * sv[...], 0.0)


_fin = pl.pallas_call(
    _fin_body,
    grid=(VP // BM,),
    in_specs=[pl.BlockSpec((BM, 128), lambda i: (i, 0))] * 4
    + [pl.BlockSpec((BM, 1), lambda i: (i, 0))],
    out_specs=pl.BlockSpec((BM, HID), lambda i: (i, 0)),
    out_shape=SDS((VP, HID), f32),
)


def _smsg_body(hl, hh, q, m0, m1, m2, m3):
    qq = q[...]
    a = hl[...]
    b = hh[...]
    m0[...] = a[:, :128] * qq
    m1[...] = a[:, 128:] * qq
    m2[...] = b[:, :128] * qq
    m3[...] = b[:, 128:] * qq


BR = 1280
_smsg = pl.pallas_call(
    _smsg_body,
    grid=(EP // BR,),
    in_specs=[
        pl.BlockSpec((BR, 256), lambda i: (i, 0)),
        pl.BlockSpec((BR, 256), lambda i: (i, 0)),
        pl.BlockSpec((BR, 1), lambda i: (i, 0)),
    ],
    out_specs=tuple(pl.BlockSpec((BR, 128), lambda i: (i, 0)) for _ in range(4)),
    out_shape=tuple(SDS((EP, 128), f32) for _ in range(4)),
)


@functools.partial(
    pl.kernel,
    out_type=(SDS((EP, 256), f32), SDS((EP, 256), f32)),
    mesh=_mesh,
    scratch_types=[
        pltpu.VMEM((CPT, 128), jnp.int32),
        pltpu.VMEM((CPT, 128), jnp.int32),
        pltpu.VMEM((128, 256), f32),
        pltpu.VMEM((64, 256), f32),
        pltpu.SemaphoreType.DMA,
        pltpu.VMEM_SHARED((EP, 256), f32),
    ],
)
def _stA(xlo, xhi, v3, e3, za, helo, hehi, vb, eb, rows, wb, sem, acc):
    c = lax.axis_index("c")
    s = lax.axis_index("s")
    pltpu.sync_copy(v3.at[s], vb)
    pltpu.sync_copy(e3.at[s], eb)
    pltpu.sync_copy(za, wb)
    for k in range(5):
        pltpu.sync_copy(wb, acc.at[pl.ds(s * 320 + k * 64, 64)])
    plsc.subcore_barrier()

    def do(xt, out):
        def step(i, _):
            pltpu.async_copy(xt.at[vb.at[i]], rows, sem).wait()
            pltpu.sync_copy(rows, acc.at[eb.at[i]], add=True)
            return _

        lax.fori_loop(0, CPT, step, None)
        plsc.subcore_barrier()
        for k in range(5):
            pltpu.sync_copy(acc.at[pl.ds(s * 320 + k * 64, 64)], wb)
            pltpu.sync_copy(wb, out.at[pl.ds(s * 320 + k * 64, 64)])

    @pl.when(c == 0)
    def _():
        do(xlo, helo)

    @pl.when(c == 1)
    def _():
        do(xhi, hehi)


@functools.partial(
    pl.kernel,
    out_type=tuple(SDS((VP, 128), f32) for _ in range(4)),
    mesh=_mesh,
    scratch_types=[
        pltpu.VMEM((CPT, 128), jnp.int32),
        pltpu.VMEM((CPT, 128), jnp.int32),
        pltpu.VMEM((128, 128), f32),
        pltpu.VMEM((128, 128), f32),
        pltpu.SemaphoreType.DMA,
        pltpu.VMEM_SHARED((VP, 128), f32),
    ],
)
def _stB(m0, m1, m2, m3, v3, e3, zb, o0, o1, o2, o3, vb, eb, rows, wb, sem, acc):
    c = lax.axis_index("c")
    s = lax.axis_index("s")
    pltpu.sync_copy(v3.at[s], vb)
    pltpu.sync_copy(e3.at[s], eb)

    def do(m, o):
        pltpu.sync_copy(zb, wb)
        for k in range(5):
            pltpu.sync_copy(wb, acc.at[pl.ds(s * 640 + k * 128, 128)])
        plsc.subcore_barrier()

        def step(i, _):
            pltpu.async_copy(m.at[eb.at[i]], rows, sem).wait()
            pltpu.sync_copy(rows, acc.at[vb.at[i]], add=True)
            return _

        lax.fori_loop(0, CPT, step, None)
        plsc.subcore_barrier()
        for k in range(5):
            pltpu.sync_copy(acc.at[pl.ds(s * 640 + k * 128, 128)], wb)
            pltpu.sync_copy(wb, o.at[pl.ds(s * 640 + k * 128, 128)])
        plsc.subcore_barrier()

    for t in range(2):
        @pl.when(c == 0)
        def _(m=(m0, m1)[t], o=(o0, o1)[t]):
            do(m, o)

        @pl.when(c == 1)
        def _(m=(m2, m3)[t], o=(o2, o3)[t]):
            do(m, o)


def kernel(x, hg, W1, b1, W2, b2):
    v = hg[0].astype(jnp.int32)
    e = hg[1].astype(jnp.int32)
    pad = NZP - NZ
    vp = jnp.concatenate([v, jnp.full((pad,), VP - 1, jnp.int32)])
    ep = jnp.concatenate([e, jnp.full((pad,), EP - 1, jnp.int32)])
    v16 = vp.reshape(NS, CPT, 128)
    e16 = ep.reshape(NS, CPT, 128)
    v32 = vp.reshape(32, DPT, 16)
    e32 = ep.reshape(32, DPT, 16)
    z = jnp.zeros((16384,), f32)
    za = z.reshape(64, 256)
    zb = z.reshape(128, 128)
    xpad = jnp.pad(x, ((0, VP - NV), (0, 0)))
    W1t = W1.T
    W2t = W2.T
    b1r = b1.reshape(1, HID)
    b2r = b2.reshape(1, HID)

    dvp, cep = _deg1(v32, e32)
    sv, dvf, cnt = _red1(dvp.reshape(32, VP), cep.reshape(32, EP))
    dep = _deg2(v32, e32, dvf.reshape(640, 16))
    q = _red2(dep.reshape(32, EP), cnt)
    sv2 = sv.reshape(VP, 1)
    q2 = q.reshape(EP, 1)

    xlo, xhi = _mm1(xpad, W1t, b1r)
    hl, hh = _stA(xlo, xhi, v16, e16, za)
    m0, m1, m2, m3 = _smsg(hl, hh, q2)
    o0, o1, o2, o3 = _stB(m0, m1, m2, m3, v16, e16, zb)
    x2lo, x2hi = _mm2(o0, o1, o2, o3, sv2, W2t, b2r)
    hl2, hh2 = _stA(x2lo, x2hi, v16, e16, za)
    n0, n1, n2, n3 = _smsg(hl2, hh2, q2)
    p0, p1, p2, p3 = _stB(n0, n1, n2, n3, v16, e16, zb)
    out = _fin(p0, p1, p2, p3, sv2)
    return out[:NV]


# speedup vs baseline: 2.2981x; 2.2981x over previous
import functools

import jax
import jax.numpy as jnp
from jax import lax
from jax.experimental import pallas as pl
from jax.experimental.pallas import tpu as pltpu
from jax.experimental.pallas import tpu_sc as plsc

NV, NE, NZ = 10000, 5000, 160000
VP, EP = 10240, 5120
NZP = 163840
CIN, HID = 256, 512
NC, NS = 2, 16
CNK = 64
NCH = 160
NCH2 = NCH // 2

f32 = jnp.float32
SDS = jax.ShapeDtypeStruct


def _deg_body(v3, e3, dv_o, ce_o, de_o, vb, eb, ones, got, wbuf, dv_sh, ce_sh, de_sh):
    c = lax.axis_index("c")
    s = lax.axis_index("s")

    @pl.when(c == 0)
    def _():
        one16 = jnp.ones((16,), f32)
        zero16 = jnp.zeros((16,), f32)

        def fill(i, _):
            ones[i] = one16
            wbuf[i] = zero16
            return _

        lax.fori_loop(0, CNK, fill, None)
        for k in range(10):
            pltpu.sync_copy(wbuf, dv_sh.at[pl.ds(s * 640 + k * 64, 64)])
        for k in range(5):
            pltpu.sync_copy(wbuf, ce_sh.at[pl.ds(s * 320 + k * 64, 64)])
            pltpu.sync_copy(wbuf, de_sh.at[pl.ds(s * 320 + k * 64, 64)])
        plsc.subcore_barrier()

        def p1(i, _):
            pltpu.sync_copy(ones, dv_sh.at[vb.at[i]], add=True)
            pltpu.sync_copy(ones, ce_sh.at[eb.at[i]], add=True)
            return _

        for h in range(2):
            pltpu.sync_copy(v3.at[s, pl.ds(h * NCH2, NCH2)], vb)
            pltpu.sync_copy(e3.at[s, pl.ds(h * NCH2, NCH2)], eb)
            lax.fori_loop(0, NCH2, p1, None)
        plsc.subcore_barrier()

        def p2(i, _):
            pltpu.sync_copy(dv_sh.at[vb.at[i]], got)
            pltpu.sync_copy(got, de_sh.at[eb.at[i]], add=True)
            return _

        for h in range(2):
            pltpu.sync_copy(v3.at[s, pl.ds(h * NCH2, NCH2)], vb)
            pltpu.sync_copy(e3.at[s, pl.ds(h * NCH2, NCH2)], eb)
            lax.fori_loop(0, NCH2, p2, None)
        plsc.subcore_barrier()
        for k in range(10):
            pltpu.sync_copy(dv_sh.at[pl.ds(s * 640 + k * 64, 64)], wbuf)
            pltpu.sync_copy(wbuf, dv_o.at[pl.ds(s * 640 + k * 64, 64)])
        for k in range(5):
            pltpu.sync_copy(ce_sh.at[pl.ds(s * 320 + k * 64, 64)], wbuf)
            pltpu.sync_copy(wbuf, ce_o.at[pl.ds(s * 320 + k * 64, 64)])
            pltpu.sync_copy(de_sh.at[pl.ds(s * 320 + k * 64, 64)], got)
            pltpu.sync_copy(got, de_o.at[pl.ds(s * 320 + k * 64, 64)])


def _red_body(dv2, ce2, des2, sv, q):
    d = dv2[...][:, 0]
    cn = ce2[...][:, 0]
    sv[...] = jnp.where(d > 0, lax.rsqrt(d), 0.0)
    cs = jnp.where(cn == 0, 1.0, cn)
    de = des2[...][:, 0] / cs
    q[...] = jnp.where(de > 0, lax.rsqrt(de), 0.0) / cs


_red = pl.pallas_call(_red_body, out_shape=(SDS((VP,), f32), SDS((EP,), f32)))


def _make_seg_body(acc_rows):
    zchunks = acc_rows // NS // CNK

    def body(t0, t1, t2, t3, g3, s3, zb, o0, o1, o2, o3, vb, eb, rows, sem, acc):
        c = lax.axis_index("c")
        s = lax.axis_index("s")

        def do(tq, out):
            pltpu.sync_copy(zb, rows)
            for k in range(zchunks):
                pltpu.sync_copy(rows, acc.at[pl.ds((s * zchunks + k) * CNK, CNK)])
            plsc.subcore_barrier()

            def step(i, _):
                pltpu.async_copy(tq.at[vb.at[i]], rows, sem).wait()
                pltpu.sync_copy(rows, acc.at[eb.at[i]], add=True)
                return _

            for h in range(2):
                pltpu.sync_copy(g3.at[s, pl.ds(h * NCH2, NCH2)], vb)
                pltpu.sync_copy(s3.at[s, pl.ds(h * NCH2, NCH2)], eb)
                lax.fori_loop(0, NCH2, step, None)
            plsc.subcore_barrier()
            for k in range(zchunks):
                base = (s * zchunks + k) * CNK
                pltpu.sync_copy(acc.at[pl.ds(base, CNK)], rows)
                pltpu.sync_copy(rows, out.at[pl.ds(base, CNK)])
            plsc.subcore_barrier()

        for t in range(2):
            @pl.when(c == 0)
            def _(tq=(t0, t1)[t], out=(o0, o1)[t]):
                do(tq, out)

            @pl.when(c == 1)
            def _(tq=(t2, t3)[t], out=(o2, o3)[t]):
                do(tq, out)

    return body


BM = 1280


def _mm1_body(x, w, b, q0, q1, q2, q3):
    xt = jnp.dot(x[...], w[...], preferred_element_type=f32) + b[...]
    q0[...] = xt[:, :128]
    q1[...] = xt[:, 128:256]
    q2[...] = xt[:, 256:384]
    q3[...] = xt[:, 384:]


_mm1 = pl.pallas_call(
    _mm1_body,
    grid=(VP // BM,),
    in_specs=[
        pl.BlockSpec((BM, CIN), lambda i: (i, 0)),
        pl.BlockSpec((CIN, HID), lambda i: (0, 0)),
        pl.BlockSpec((1, HID), lambda i: (0, 0)),
    ],
    out_specs=tuple(pl.BlockSpec((BM, 128), lambda i: (i, 0)) for _ in range(4)),
    out_shape=tuple(SDS((VP, 128), f32) for _ in range(4)),
)


def _mm2_body(o0, o1, o2, o3, sv, w, b, q0, q1, q2, q3):
    h = jnp.concatenate([o0[...], o1[...], o2[...], o3[...]], axis=1)
    h = jnp.maximum(h * sv[...], 0.0)
    xt = jnp.dot(h, w[...], preferred_element_type=f32) + b[...]
    q0[...] = xt[:, :128]
    q1[...] = xt[:, 128:256]
    q2[...] = xt[:, 256:384]
    q3[...] = xt[:, 384:]


_mm2 = pl.pallas_call(
    _mm2_body,
    grid=(VP // BM,),
    in_specs=[pl.BlockSpec((BM, 128), lambda i: (i, 0))] * 4
    + [
        pl.BlockSpec((BM, 1), lambda i: (i, 0)),
        pl.BlockSpec((HID, HID), lambda i: (0, 0)),
        pl.BlockSpec((1, HID), lambda i: (0, 0)),
    ],
    out_specs=tuple(pl.BlockSpec((BM, 128), lambda i: (i, 0)) for _ in range(4)),
    out_shape=tuple(SDS((VP, 128), f32) for _ in range(4)),
)


def _fin_body(o0, o1, o2, o3, sv, out):
    h = jnp.concatenate([o0[...], o1[...], o2[...], o3[...]], axis=1)
    out[...] = jnp.maximum(h * sv[...], 0.0)


_fin = pl.pallas_call(
    _fin_body,
    grid=(VP // BM,),
    in_specs=[pl.BlockSpec((BM, 128), lambda i: (i, 0))] * 4
    + [pl.BlockSpec((BM, 1), lambda i: (i, 0))],
    out_specs=pl.BlockSpec((BM, HID), lambda i: (i, 0)),
    out_shape=SDS((VP, HID), f32),
)


def _smsg_body(h0, h1, h2, h3, q, m0, m1, m2, m3):
    qq = q[...]
    m0[...] = h0[...] * qq
    m1[...] = h1[...] * qq
    m2[...] = h2[...] * qq
    m3[...] = h3[...] * qq


_smsg = pl.pallas_call(
    _smsg_body,
    grid=(4,),
    in_specs=[pl.BlockSpec((EP // 4, 128), lambda i: (i, 0))] * 4
    + [pl.BlockSpec((EP // 4, 1), lambda i: (i, 0))],
    out_specs=tuple(pl.BlockSpec((EP // 4, 128), lambda i: (i, 0)) for _ in range(4)),
    out_shape=tuple(SDS((EP, 128), f32) for _ in range(4)),
)


@functools.lru_cache(maxsize=1)
def _build_sc():
    mesh = plsc.VectorSubcoreMesh(
        core_axis_name="c", subcore_axis_name="s", num_cores=NC, num_subcores=NS
    )
    deg = pl.kernel(
        _deg_body,
        out_type=(SDS((VP, 16), f32), SDS((EP, 16), f32), SDS((EP, 16), f32)),
        mesh=mesh,
        scratch_types=[
            pltpu.VMEM((NCH2, CNK), jnp.int32),
            pltpu.VMEM((NCH2, CNK), jnp.int32),
            pltpu.VMEM((CNK, 16), f32),
            pltpu.VMEM((CNK, 16), f32),
            pltpu.VMEM((CNK, 16), f32),
            pltpu.VMEM_SHARED((VP, 16), f32),
            pltpu.VMEM_SHARED((EP, 16), f32),
            pltpu.VMEM_SHARED((EP, 16), f32),
        ],
    )

    def seg(acc_rows):
        return pl.kernel(
            _make_seg_body(acc_rows),
            out_type=tuple(SDS((acc_rows, 128), f32) for _ in range(4)),
            mesh=mesh,
            scratch_types=[
                pltpu.VMEM((NCH2, CNK), jnp.int32),
                pltpu.VMEM((NCH2, CNK), jnp.int32),
                pltpu.VMEM((CNK, 128), f32),
                pltpu.SemaphoreType.DMA,
                pltpu.VMEM_SHARED((acc_rows, 128), f32),
            ],
        )

    return deg, seg(EP), seg(VP)


def kernel(x, hg, W1, b1, W2, b2):
    _deg, _stA, _stB = _build_sc()
    v = hg[0].astype(jnp.int32)
    e = hg[1].astype(jnp.int32)
    pad = NZP - NZ
    vp = jnp.concatenate([v, jnp.full((pad,), VP - 1, jnp.int32)])
    ep = jnp.concatenate([e, jnp.full((pad,), EP - 1, jnp.int32)])
    v16 = vp.reshape(NS, NCH, CNK)
    e16 = ep.reshape(NS, NCH, CNK)
    zb = jnp.zeros((CNK, 128), f32)
    xpad = jnp.pad(x, ((0, VP - NV), (0, 0)))
    W1t = W1.T
    W2t = W2.T
    b1r = b1.reshape(1, HID)
    b2r = b2.reshape(1, HID)

    dv2, ce2, de2 = _deg(v16, e16)
    sv, q = _red(dv2, ce2, de2)
    sv2 = sv.reshape(VP, 1)
    q2 = q.reshape(EP, 1)

    x0, x1, x2, x3 = _mm1(xpad, W1t, b1r)
    h0, h1, h2, h3 = _stA(x0, x1, x2, x3, v16, e16, zb)
    m0, m1, m2, m3 = _smsg(h0, h1, h2, h3, q2)
    o0, o1, o2, o3 = _stB(m0, m1, m2, m3, e16, v16, zb)
    y0, y1, y2, y3 = _mm2(o0, o1, o2, o3, sv2, W2t, b2r)
    g0, g1, g2, g3 = _stA(y0, y1, y2, y3, v16, e16, zb)
    n0, n1, n2, n3 = _smsg(g0, g1, g2, g3, q2)
    p0, p1, p2, p3 = _stB(n0, n1, n2, n3, e16, v16, zb)
    out = _fin(p0, p1, p2, p3, sv2)
    return out[:NV]


# speedup vs baseline: 3.0021x; 1.3063x over previous
import functools

import jax
import jax.numpy as jnp
from jax import lax
from jax.experimental import pallas as pl
from jax.experimental.pallas import tpu as pltpu
from jax.experimental.pallas import tpu_sc as plsc

NV, NE, NZ = 10000, 5000, 160000
VP, EP = 10240, 5120
NZP = 163840
CIN, HID = 256, 512
NC, NS = 2, 16
CNK = 64
NCH = 160
NCH2 = NCH // 2

f32 = jnp.float32
SDS = jax.ShapeDtypeStruct


def _deg_body(v3, e3, dv_o, ce_o, de_o, vb, eb, ones, got, wbuf, dv_sh, ce_sh, de_sh):
    c = lax.axis_index("c")
    s = lax.axis_index("s")

    @pl.when(c == 0)
    def _():
        one16 = jnp.ones((16,), f32)
        zero16 = jnp.zeros((16,), f32)

        def fill(i, _):
            ones[i] = one16
            wbuf[i] = zero16
            return _

        lax.fori_loop(0, CNK, fill, None)
        for k in range(10):
            pltpu.sync_copy(wbuf, dv_sh.at[pl.ds(s * 640 + k * 64, 64)])
        for k in range(5):
            pltpu.sync_copy(wbuf, ce_sh.at[pl.ds(s * 320 + k * 64, 64)])
            pltpu.sync_copy(wbuf, de_sh.at[pl.ds(s * 320 + k * 64, 64)])
        plsc.subcore_barrier()

        def p1(i, _):
            pltpu.sync_copy(ones, dv_sh.at[vb.at[i]], add=True)
            pltpu.sync_copy(ones, ce_sh.at[eb.at[i]], add=True)
            return _

        for h in range(2):
            pltpu.sync_copy(v3.at[s, pl.ds(h * NCH2, NCH2)], vb)
            pltpu.sync_copy(e3.at[s, pl.ds(h * NCH2, NCH2)], eb)
            lax.fori_loop(0, NCH2, p1, None)
        plsc.subcore_barrier()

        def p2(i, _):
            pltpu.sync_copy(dv_sh.at[vb.at[i]], got)
            pltpu.sync_copy(got, de_sh.at[eb.at[i]], add=True)
            return _

        for h in range(2):
            pltpu.sync_copy(v3.at[s, pl.ds(h * NCH2, NCH2)], vb)
            pltpu.sync_copy(e3.at[s, pl.ds(h * NCH2, NCH2)], eb)
            lax.fori_loop(0, NCH2, p2, None)
        plsc.subcore_barrier()
        for k in range(10):
            pltpu.sync_copy(dv_sh.at[pl.ds(s * 640 + k * 64, 64)], wbuf)
            pltpu.sync_copy(wbuf, dv_o.at[pl.ds(s * 640 + k * 64, 64)])
        for k in range(5):
            pltpu.sync_copy(ce_sh.at[pl.ds(s * 320 + k * 64, 64)], wbuf)
            pltpu.sync_copy(wbuf, ce_o.at[pl.ds(s * 320 + k * 64, 64)])
            pltpu.sync_copy(de_sh.at[pl.ds(s * 320 + k * 64, 64)], got)
            pltpu.sync_copy(got, de_o.at[pl.ds(s * 320 + k * 64, 64)])


def _red_body(dv2, ce2, des2, sv, q):
    d = dv2[...][:, 0]
    cn = ce2[...][:, 0]
    sv[...] = jnp.where(d > 0, lax.rsqrt(d), 0.0)
    cs = jnp.where(cn == 0, 1.0, cn)
    de = des2[...][:, 0] / cs
    q[...] = jnp.where(de > 0, lax.rsqrt(de), 0.0) / cs


_red = pl.pallas_call(_red_body, out_shape=(SDS((VP,), f32), SDS((EP,), f32)))


def _make_seg_body(acc_rows):
    zchunks = acc_rows // NS // CNK

    def body(t0, t1, t2, t3, g3, s3, zb, o0, o1, o2, o3, vb, eb, bufa, bufb,
             sema, semb, acc):
        c = lax.axis_index("c")
        s = lax.axis_index("s")

        def do(tq, out):
            pltpu.sync_copy(zb, bufa)
            for k in range(zchunks):
                pltpu.sync_copy(bufa, acc.at[pl.ds((s * zchunks + k) * CNK, CNK)])
            plsc.subcore_barrier()

            def pair(j, _):
                i0 = 2 * j
                pltpu.async_copy(tq.at[vb.at[i0 + 1]], bufb, semb)
                pltpu.make_async_copy(zb, bufa, sema).wait()
                pltpu.sync_copy(bufa, acc.at[eb.at[i0]], add=True)

                @pl.when(j + 1 < NCH2 // 2)
                def _():
                    pltpu.async_copy(tq.at[vb.at[i0 + 2]], bufa, sema)

                pltpu.make_async_copy(zb, bufb, semb).wait()
                pltpu.sync_copy(bufb, acc.at[eb.at[i0 + 1]], add=True)
                return _

            for h in range(2):
                pltpu.sync_copy(g3.at[s, pl.ds(h * NCH2, NCH2)], vb)
                pltpu.sync_copy(s3.at[s, pl.ds(h * NCH2, NCH2)], eb)
                pltpu.async_copy(tq.at[vb.at[0]], bufa, sema)
                lax.fori_loop(0, NCH2 // 2, pair, None)
            plsc.subcore_barrier()
            for k in range(zchunks):
                base = (s * zchunks + k) * CNK
                pltpu.sync_copy(acc.at[pl.ds(base, CNK)], bufa)
                pltpu.sync_copy(bufa, out.at[pl.ds(base, CNK)])
            plsc.subcore_barrier()

        for t in range(2):
            @pl.when(c == 0)
            def _(tq=(t0, t1)[t], out=(o0, o1)[t]):
                do(tq, out)

            @pl.when(c == 1)
            def _(tq=(t2, t3)[t], out=(o2, o3)[t]):
                do(tq, out)

    return body


BM = 1280


def _mm1_body(x, w, b, q0, q1, q2, q3):
    xt = jnp.dot(x[...], w[...], preferred_element_type=f32) + b[...]
    q0[...] = xt[:, :128]
    q1[...] = xt[:, 128:256]
    q2[...] = xt[:, 256:384]
    q3[...] = xt[:, 384:]


_mm1 = pl.pallas_call(
    _mm1_body,
    grid=(VP // BM,),
    in_specs=[
        pl.BlockSpec((BM, CIN), lambda i: (i, 0)),
        pl.BlockSpec((CIN, HID), lambda i: (0, 0)),
        pl.BlockSpec((1, HID), lambda i: (0, 0)),
    ],
    out_specs=tuple(pl.BlockSpec((BM, 128), lambda i: (i, 0)) for _ in range(4)),
    out_shape=tuple(SDS((VP, 128), f32) for _ in range(4)),
)


def _mm2_body(o0, o1, o2, o3, sv, w, b, q0, q1, q2, q3):
    h = jnp.concatenate([o0[...], o1[...], o2[...], o3[...]], axis=1)
    h = jnp.maximum(h * sv[...], 0.0)
    xt = jnp.dot(h, w[...], preferred_element_type=f32) + b[...]
    q0[...] = xt[:, :128]
    q1[...] = xt[:, 128:256]
    q2[...] = xt[:, 256:384]
    q3[...] = xt[:, 384:]


_mm2 = pl.pallas_call(
    _mm2_body,
    grid=(VP // BM,),
    in_specs=[pl.BlockSpec((BM, 128), lambda i: (i, 0))] * 4
    + [
        pl.BlockSpec((BM, 1), lambda i: (i, 0)),
        pl.BlockSpec((HID, HID), lambda i: (0, 0)),
        pl.BlockSpec((1, HID), lambda i: (0, 0)),
    ],
    out_specs=tuple(pl.BlockSpec((BM, 128), lambda i: (i, 0)) for _ in range(4)),
    out_shape=tuple(SDS((VP, 128), f32) for _ in range(4)),
)


def _fin_body(o0, o1, o2, o3, sv, out):
    h = jnp.concatenate([o0[...], o1[...], o2[...], o3[...]], axis=1)
    out[...] = jnp.maximum(h * sv[...], 0.0)


_fin = pl.pallas_call(
    _fin_body,
    grid=(VP // BM,),
    in_specs=[pl.BlockSpec((BM, 128), lambda i: (i, 0))] * 4
    + [pl.BlockSpec((BM, 1), lambda i: (i, 0))],
    out_specs=pl.BlockSpec((BM, HID), lambda i: (i, 0)),
    out_shape=SDS((VP, HID), f32),
)


def _smsg_body(h0, h1, h2, h3, q, m0, m1, m2, m3):
    qq = q[...]
    m0[...] = h0[...] * qq
    m1[...] = h1[...] * qq
    m2[...] = h2[...] * qq
    m3[...] = h3[...] * qq


_smsg = pl.pallas_call(
    _smsg_body,
    grid=(4,),
    in_specs=[pl.BlockSpec((EP // 4, 128), lambda i: (i, 0))] * 4
    + [pl.BlockSpec((EP // 4, 1), lambda i: (i, 0))],
    out_specs=tuple(pl.BlockSpec((EP // 4, 128), lambda i: (i, 0)) for _ in range(4)),
    out_shape=tuple(SDS((EP, 128), f32) for _ in range(4)),
)


@functools.lru_cache(maxsize=1)
def _build_sc():
    mesh = plsc.VectorSubcoreMesh(
        core_axis_name="c", subcore_axis_name="s", num_cores=NC, num_subcores=NS
    )
    deg = pl.kernel(
        _deg_body,
        out_type=(SDS((VP, 16), f32), SDS((EP, 16), f32), SDS((EP, 16), f32)),
        mesh=mesh,
        scratch_types=[
            pltpu.VMEM((NCH2, CNK), jnp.int32),
            pltpu.VMEM((NCH2, CNK), jnp.int32),
            pltpu.VMEM((CNK, 16), f32),
            pltpu.VMEM((CNK, 16), f32),
            pltpu.VMEM((CNK, 16), f32),
            pltpu.VMEM_SHARED((VP, 16), f32),
            pltpu.VMEM_SHARED((EP, 16), f32),
            pltpu.VMEM_SHARED((EP, 16), f32),
        ],
    )

    def seg(acc_rows):
        return pl.kernel(
            _make_seg_body(acc_rows),
            out_type=tuple(SDS((acc_rows, 128), f32) for _ in range(4)),
            mesh=mesh,
            scratch_types=[
                pltpu.VMEM((NCH2, CNK), jnp.int32),
                pltpu.VMEM((NCH2, CNK), jnp.int32),
                pltpu.VMEM((CNK, 128), f32),
                pltpu.VMEM((CNK, 128), f32),
                pltpu.SemaphoreType.DMA,
                pltpu.SemaphoreType.DMA,
                pltpu.VMEM_SHARED((acc_rows, 128), f32),
            ],
        )

    return deg, seg(EP), seg(VP)


def kernel(x, hg, W1, b1, W2, b2):
    _deg, _stA, _stB = _build_sc()
    v = hg[0].astype(jnp.int32)
    e = hg[1].astype(jnp.int32)
    pad = NZP - NZ
    vp = jnp.concatenate([v, jnp.full((pad,), VP - 1, jnp.int32)])
    ep = jnp.concatenate([e, jnp.full((pad,), EP - 1, jnp.int32)])
    v16 = vp.reshape(NS, NCH, CNK)
    e16 = ep.reshape(NS, NCH, CNK)
    zb = jnp.zeros((CNK, 128), f32)
    xpad = jnp.pad(x, ((0, VP - NV), (0, 0)))
    W1t = W1.T
    W2t = W2.T
    b1r = b1.reshape(1, HID)
    b2r = b2.reshape(1, HID)

    dv2, ce2, de2 = _deg(v16, e16)
    sv, q = _red(dv2, ce2, de2)
    sv2 = sv.reshape(VP, 1)
    q2 = q.reshape(EP, 1)

    x0, x1, x2, x3 = _mm1(xpad, W1t, b1r)
    h0, h1, h2, h3 = _stA(x0, x1, x2, x3, v16, e16, zb)
    m0, m1, m2, m3 = _smsg(h0, h1, h2, h3, q2)
    o0, o1, o2, o3 = _stB(m0, m1, m2, m3, e16, v16, zb)
    y0, y1, y2, y3 = _mm2(o0, o1, o2, o3, sv2, W2t, b2r)
    g0, g1, g2, g3 = _stA(y0, y1, y2, y3, v16, e16, zb)
    n0, n1, n2, n3 = _smsg(g0, g1, g2, g3, q2)
    p0, p1, p2, p3 = _stB(n0, n1, n2, n3, e16, v16, zb)
    out = _fin(p0, p1, p2, p3, sv2)
    return out[:NV]


# speedup vs baseline: 3.0955x; 1.0311x over previous
import functools

import jax
import jax.numpy as jnp
from jax import lax
from jax.experimental import pallas as pl
from jax.experimental.pallas import tpu as pltpu
from jax.experimental.pallas import tpu_sc as plsc

NV, NE, NZ = 10000, 5000, 160000
VP, EP = 10240, 5120
NZP = 163840
CIN, HID = 256, 512
NC, NS = 2, 16
CNK = 64
NCH = 160
NCH2 = NCH // 2

f32 = jnp.float32
SDS = jax.ShapeDtypeStruct


def _deg_body(v3, e3, dv_o, ce_o, de_o, vb, eb, ones, got, wbuf, dv_sh, ce_sh, de_sh):
    c = lax.axis_index("c")
    s = lax.axis_index("s")

    @pl.when(c == 0)
    def _():
        one16 = jnp.ones((16,), f32)
        zero16 = jnp.zeros((16,), f32)

        def fill(i, _):
            ones[i] = one16
            wbuf[i] = zero16
            return _

        lax.fori_loop(0, CNK, fill, None)
        for k in range(10):
            pltpu.sync_copy(wbuf, dv_sh.at[pl.ds(s * 640 + k * 64, 64)])
        for k in range(5):
            pltpu.sync_copy(wbuf, ce_sh.at[pl.ds(s * 320 + k * 64, 64)])
            pltpu.sync_copy(wbuf, de_sh.at[pl.ds(s * 320 + k * 64, 64)])
        plsc.subcore_barrier()

        def p1(i, _):
            pltpu.sync_copy(ones, dv_sh.at[vb.at[i]], add=True)
            pltpu.sync_copy(ones, ce_sh.at[eb.at[i]], add=True)
            return _

        for h in range(2):
            pltpu.sync_copy(v3.at[s, pl.ds(h * NCH2, NCH2)], vb)
            pltpu.sync_copy(e3.at[s, pl.ds(h * NCH2, NCH2)], eb)
            lax.fori_loop(0, NCH2, p1, None)
        plsc.subcore_barrier()

        def p2(i, _):
            pltpu.sync_copy(dv_sh.at[vb.at[i]], got)
            pltpu.sync_copy(got, de_sh.at[eb.at[i]], add=True)
            return _

        for h in range(2):
            pltpu.sync_copy(v3.at[s, pl.ds(h * NCH2, NCH2)], vb)
            pltpu.sync_copy(e3.at[s, pl.ds(h * NCH2, NCH2)], eb)
            lax.fori_loop(0, NCH2, p2, None)
        plsc.subcore_barrier()
        for k in range(10):
            pltpu.sync_copy(dv_sh.at[pl.ds(s * 640 + k * 64, 64)], wbuf)
            pltpu.sync_copy(wbuf, dv_o.at[pl.ds(s * 640 + k * 64, 64)])
        for k in range(5):
            pltpu.sync_copy(ce_sh.at[pl.ds(s * 320 + k * 64, 64)], wbuf)
            pltpu.sync_copy(wbuf, ce_o.at[pl.ds(s * 320 + k * 64, 64)])
            pltpu.sync_copy(de_sh.at[pl.ds(s * 320 + k * 64, 64)], got)
            pltpu.sync_copy(got, de_o.at[pl.ds(s * 320 + k * 64, 64)])


def _red_body(dv2, ce2, des2, sv, q):
    d = dv2[...][:, 0]
    cn = ce2[...][:, 0]
    sv[...] = jnp.where(d > 0, lax.rsqrt(d), 0.0)
    cs = jnp.where(cn == 0, 1.0, cn)
    de = des2[...][:, 0] / cs
    q[...] = jnp.where(de > 0, lax.rsqrt(de), 0.0) / cs


_red = pl.pallas_call(_red_body, out_shape=(SDS((VP,), f32), SDS((EP,), f32)))


def _make_seg_body(acc_rows, cnk):
    zchunks = acc_rows // NS // 64
    nch = NZP // NS // cnk
    nhalf = nch // 2

    def body(t0, t1, t2, t3, g3, s3, zb, o0, o1, o2, o3, vb, eb, bufa, bufb,
             sema, semb, acc):
        c = lax.axis_index("c")
        s = lax.axis_index("s")

        def do(tq, out):
            pltpu.sync_copy(zb, bufa)
            for k in range(zchunks):
                pltpu.sync_copy(
                    bufa.at[pl.ds(0, 64)],
                    acc.at[pl.ds((s * zchunks + k) * 64, 64)],
                )
            plsc.subcore_barrier()

            def pair(j, _):
                i0 = 2 * j
                pltpu.async_copy(tq.at[vb.at[i0 + 1]], bufb, semb)
                pltpu.make_async_copy(zb, bufa, sema).wait()
                pltpu.sync_copy(bufa, acc.at[eb.at[i0]], add=True)

                @pl.when(j + 1 < nhalf // 2)
                def _():
                    pltpu.async_copy(tq.at[vb.at[i0 + 2]], bufa, sema)

                pltpu.make_async_copy(zb, bufb, semb).wait()
                pltpu.sync_copy(bufb, acc.at[eb.at[i0 + 1]], add=True)
                return _

            for h in range(2):
                pltpu.sync_copy(g3.at[s, pl.ds(h * nhalf, nhalf)], vb)
                pltpu.sync_copy(s3.at[s, pl.ds(h * nhalf, nhalf)], eb)
                pltpu.async_copy(tq.at[vb.at[0]], bufa, sema)
                lax.fori_loop(0, nhalf // 2, pair, None)
            plsc.subcore_barrier()
            for k in range(zchunks):
                base = (s * zchunks + k) * 64
                pltpu.sync_copy(acc.at[pl.ds(base, 64)], bufa.at[pl.ds(0, 64)])
                pltpu.sync_copy(bufa.at[pl.ds(0, 64)], out.at[pl.ds(base, 64)])
            plsc.subcore_barrier()

        for t in range(2):
            @pl.when(c == 0)
            def _(tq=(t0, t1)[t], out=(o0, o1)[t]):
                do(tq, out)

            @pl.when(c == 1)
            def _(tq=(t2, t3)[t], out=(o2, o3)[t]):
                do(tq, out)

    return body


BM = 1280


def _mm1_body(x, w, b, q0, q1, q2, q3):
    xt = jnp.dot(x[...], w[...], preferred_element_type=f32) + b[...]
    q0[...] = xt[:, :128]
    q1[...] = xt[:, 128:256]
    q2[...] = xt[:, 256:384]
    q3[...] = xt[:, 384:]


_mm1 = pl.pallas_call(
    _mm1_body,
    grid=(VP // BM,),
    in_specs=[
        pl.BlockSpec((BM, CIN), lambda i: (i, 0)),
        pl.BlockSpec((CIN, HID), lambda i: (0, 0)),
        pl.BlockSpec((1, HID), lambda i: (0, 0)),
    ],
    out_specs=tuple(pl.BlockSpec((BM, 128), lambda i: (i, 0)) for _ in range(4)),
    out_shape=tuple(SDS((VP, 128), f32) for _ in range(4)),
)


def _mm2_body(o0, o1, o2, o3, sv, w, b, q0, q1, q2, q3):
    h = jnp.concatenate([o0[...], o1[...], o2[...], o3[...]], axis=1)
    h = jnp.maximum(h * sv[...], 0.0)
    xt = jnp.dot(h, w[...], preferred_element_type=f32) + b[...]
    q0[...] = xt[:, :128]
    q1[...] = xt[:, 128:256]
    q2[...] = xt[:, 256:384]
    q3[...] = xt[:, 384:]


_mm2 = pl.pallas_call(
    _mm2_body,
    grid=(VP // BM,),
    in_specs=[pl.BlockSpec((BM, 128), lambda i: (i, 0))] * 4
    + [
        pl.BlockSpec((BM, 1), lambda i: (i, 0)),
        pl.BlockSpec((HID, HID), lambda i: (0, 0)),
        pl.BlockSpec((1, HID), lambda i: (0, 0)),
    ],
    out_specs=tuple(pl.BlockSpec((BM, 128), lambda i: (i, 0)) for _ in range(4)),
    out_shape=tuple(SDS((VP, 128), f32) for _ in range(4)),
)


def _fin_body(o0, o1, o2, o3, sv, out):
    h = jnp.concatenate([o0[...], o1[...], o2[...], o3[...]], axis=1)
    out[...] = jnp.maximum(h * sv[...], 0.0)


_fin = pl.pallas_call(
    _fin_body,
    grid=(VP // BM,),
    in_specs=[pl.BlockSpec((BM, 128), lambda i: (i, 0))] * 4
    + [pl.BlockSpec((BM, 1), lambda i: (i, 0))],
    out_specs=pl.BlockSpec((BM, HID), lambda i: (i, 0)),
    out_shape=SDS((VP, HID), f32),
)


def _smsg_body(h0, h1, h2, h3, q, m0, m1, m2, m3):
    qq = q[...]
    m0[...] = h0[...] * qq
    m1[...] = h1[...] * qq
    m2[...] = h2[...] * qq
    m3[...] = h3[...] * qq


_smsg = pl.pallas_call(
    _smsg_body,
    grid=(4,),
    in_specs=[pl.BlockSpec((EP // 4, 128), lambda i: (i, 0))] * 4
    + [pl.BlockSpec((EP // 4, 1), lambda i: (i, 0))],
    out_specs=tuple(pl.BlockSpec((EP // 4, 128), lambda i: (i, 0)) for _ in range(4)),
    out_shape=tuple(SDS((EP, 128), f32) for _ in range(4)),
)


@functools.lru_cache(maxsize=1)
def _build_sc():
    mesh = plsc.VectorSubcoreMesh(
        core_axis_name="c", subcore_axis_name="s", num_cores=NC, num_subcores=NS
    )
    deg = pl.kernel(
        _deg_body,
        out_type=(SDS((VP, 16), f32), SDS((EP, 16), f32), SDS((EP, 16), f32)),
        mesh=mesh,
        scratch_types=[
            pltpu.VMEM((NCH2, CNK), jnp.int32),
            pltpu.VMEM((NCH2, CNK), jnp.int32),
            pltpu.VMEM((CNK, 16), f32),
            pltpu.VMEM((CNK, 16), f32),
            pltpu.VMEM((CNK, 16), f32),
            pltpu.VMEM_SHARED((VP, 16), f32),
            pltpu.VMEM_SHARED((EP, 16), f32),
            pltpu.VMEM_SHARED((EP, 16), f32),
        ],
    )

    def seg(acc_rows, cnk):
        nhalf = NZP // NS // cnk // 2
        return pl.kernel(
            _make_seg_body(acc_rows, cnk),
            out_type=tuple(SDS((acc_rows, 128), f32) for _ in range(4)),
            mesh=mesh,
            scratch_types=[
                pltpu.VMEM((nhalf, cnk), jnp.int32),
                pltpu.VMEM((nhalf, cnk), jnp.int32),
                pltpu.VMEM((cnk, 128), f32),
                pltpu.VMEM((cnk, 128), f32),
                pltpu.SemaphoreType.DMA,
                pltpu.SemaphoreType.DMA,
                pltpu.VMEM_SHARED((acc_rows, 128), f32),
            ],
        )

    return deg, seg(EP, 128), seg(VP, 64)


def kernel(x, hg, W1, b1, W2, b2):
    _deg, _stA, _stB = _build_sc()
    v = hg[0].astype(jnp.int32)
    e = hg[1].astype(jnp.int32)
    pad = NZP - NZ
    vp = jnp.concatenate([v, jnp.full((pad,), VP - 1, jnp.int32)])
    ep = jnp.concatenate([e, jnp.full((pad,), EP - 1, jnp.int32)])
    v16 = vp.reshape(NS, NCH, CNK)
    e16 = ep.reshape(NS, NCH, CNK)
    vA = vp.reshape(NS, NCH // 2, 128)
    eA = ep.reshape(NS, NCH // 2, 128)
    zba = jnp.zeros((128, 128), f32)
    zbb = jnp.zeros((64, 128), f32)
    xpad = jnp.pad(x, ((0, VP - NV), (0, 0)))
    W1t = W1.T
    W2t = W2.T
    b1r = b1.reshape(1, HID)
    b2r = b2.reshape(1, HID)

    dv2, ce2, de2 = _deg(v16, e16)
    sv, q = _red(dv2, ce2, de2)
    sv2 = sv.reshape(VP, 1)
    q2 = q.reshape(EP, 1)

    x0, x1, x2, x3 = _mm1(xpad, W1t, b1r)
    h0, h1, h2, h3 = _stA(x0, x1, x2, x3, vA, eA, zba)
    m0, m1, m2, m3 = _smsg(h0, h1, h2, h3, q2)
    o0, o1, o2, o3 = _stB(m0, m1, m2, m3, e16, v16, zbb)
    y0, y1, y2, y3 = _mm2(o0, o1, o2, o3, sv2, W2t, b2r)
    g0, g1, g2, g3 = _stA(y0, y1, y2, y3, vA, eA, zba)
    n0, n1, n2, n3 = _smsg(g0, g1, g2, g3, q2)
    p0, p1, p2, p3 = _stB(n0, n1, n2, n3, e16, v16, zbb)
    out = _fin(p0, p1, p2, p3, sv2)
    return out[:NV]
